# baseline (speedup 1.0000x reference)
import functools

import jax
import jax.numpy as jnp
from jax import lax
from jax.experimental import pallas as pl
from jax.experimental.pallas import tpu as pltpu
from jax.experimental.pallas import tpu_sc as plsc

_VOCAB = 1000000
_EMB = 64
_HID = 64
_B = 4096
_L = 50
_N = _B * _L

_CHUNK = 128


def _make_sc_gather():
  info = plsc.get_sparse_core_info()
  nw = info.num_cores * info.num_subcores
  per_w = _N // nw
  n_chunks = per_w // _CHUNK

  mesh = plsc.VectorSubcoreMesh(core_axis_name="c", subcore_axis_name="s")

  @functools.partial(
      pl.kernel,
      mesh=mesh,
      out_type=jax.ShapeDtypeStruct((_N, _EMB), jnp.float32),
      scratch_types=[
          pltpu.VMEM((n_chunks, _CHUNK), jnp.int32),
          pltpu.VMEM((_CHUNK, _EMB), jnp.float32),
          pltpu.SemaphoreType.DMA,
      ],
      compiler_params=pltpu.CompilerParams(use_tc_tiling_on_sc=False),
  )
  def gather_k(idx_hbm, table_hbm, out_hbm, idx_v, buf_v, sem):
    wid = lax.axis_index("s") * info.num_cores + lax.axis_index("c")
    base = wid * per_w
    pltpu.sync_copy(idx_hbm.at[wid], idx_v)

    def body(c, carry):
      pltpu.async_copy(table_hbm.at[idx_v.at[c]], buf_v, sem).wait()
      pltpu.sync_copy(buf_v, out_hbm.at[pl.ds(base + c * _CHUNK, _CHUNK)])
      return carry

    lax.fori_loop(0, n_chunks, body, 0)

  return gather_k, nw, n_chunks


def _gru_body(src_ref, x_ref, wxr_ref, wxz_ref, wxn_ref,
              whr_ref, whz_ref, whn_ref,
              bir_ref, biz_ref, bin_ref, bhr_ref, bhz_ref, bhn_ref,
              h_ref):
  t = pl.program_id(0)

  @pl.when(t == 0)
  def _():
    h_ref[...] = jnp.zeros_like(h_ref)

  x = x_ref[0]
  h = h_ref[...]
  srow = src_ref[0]

  m_row = (srow != 0).astype(jnp.float32)
  m_col = lax.dot_general(
      m_row, jnp.ones((1, 1), jnp.float32),
      (((0,), (0,)), ((), ())), preferred_element_type=jnp.float32)

  def mm(a, w_ref):
    return lax.dot_general(
        a, w_ref[...], (((1,), (0,)), ((), ())),
        preferred_element_type=jnp.float32)

  gxr = m_col * mm(x, wxr_ref) + bir_ref[...]
  gxz = m_col * mm(x, wxz_ref) + biz_ref[...]
  gxn = m_col * mm(x, wxn_ref) + bin_ref[...]
  ghr = mm(h, whr_ref) + bhr_ref[...]
  ghz = mm(h, whz_ref) + bhz_ref[...]
  ghn = mm(h, whn_ref) + bhn_ref[...]

  r = jax.nn.sigmoid(gxr + ghr)
  z = jax.nn.sigmoid(gxz + ghz)
  n = jnp.tanh(gxn + r * ghn)
  h_ref[...] = (1.0 - z) * n + z * h


def _run_gru(src3, xs, W_ih, W_hh, b_ih, b_hh):
  wih_t = W_ih.T
  whh_t = W_hh.T
  wxr, wxz, wxn = (wih_t[:, :_HID], wih_t[:, _HID:2 * _HID],
                   wih_t[:, 2 * _HID:])
  whr, whz, whn = (whh_t[:, :_HID], whh_t[:, _HID:2 * _HID],
                   whh_t[:, 2 * _HID:])
  bir, biz, bin_ = (b_ih[:_HID].reshape(1, _HID),
                    b_ih[_HID:2 * _HID].reshape(1, _HID),
                    b_ih[2 * _HID:].reshape(1, _HID))
  bhr, bhz, bhn = (b_hh[:_HID].reshape(1, _HID),
                   b_hh[_HID:2 * _HID].reshape(1, _HID),
                   b_hh[2 * _HID:].reshape(1, _HID))

  full = lambda shape: pl.BlockSpec(shape, lambda t: (0,) * len(shape))
  grid_spec = pltpu.PrefetchScalarGridSpec(
      num_scalar_prefetch=0,
      grid=(_L,),
      in_specs=[
          pl.BlockSpec((1, 1, _B), lambda t: (t, 0, 0)),
          pl.BlockSpec((1, _B, _EMB), lambda t: (t, 0, 0)),
          full((_EMB, _HID)), full((_EMB, _HID)), full((_EMB, _HID)),
          full((_HID, _HID)), full((_HID, _HID)), full((_HID, _HID)),
          full((1, _HID)), full((1, _HID)), full((1, _HID)),
          full((1, _HID)), full((1, _HID)), full((1, _HID)),
      ],
      out_specs=pl.BlockSpec((_B, _HID), lambda t: (0, 0)),
  )
  h = pl.pallas_call(
      _gru_body,
      grid_spec=grid_spec,
      out_shape=jax.ShapeDtypeStruct((_B, _HID), jnp.float32),
  )(src3, xs, wxr, wxz, wxn, whr, whz, whn, bir, biz, bin_, bhr, bhz, bhn)
  return h


@jax.jit
def kernel(src, emb_table, W_ih, W_hh, b_ih, b_hh):
  gather_k, nw, n_chunks = _make_sc_gather()
  src_t = src.T
  idx = src_t.reshape(nw, n_chunks, _CHUNK)
  gathered = gather_k(idx, emb_table)
  xs = gathered.reshape(_L, _B, _EMB)
  src3 = src_t.reshape(_L, 1, _B)
  h = _run_gru(src3, xs, W_ih, W_hh, b_ih, b_hh)
  return h[None, :, :]


# speedup vs baseline: 1.0339x; 1.0339x over previous
import functools

import jax
import jax.numpy as jnp
from jax import lax
from jax.experimental import pallas as pl
from jax.experimental.pallas import tpu as pltpu
from jax.experimental.pallas import tpu_sc as plsc

_VOCAB = 1000000
_EMB = 64
_HID = 64
_B = 4096
_L = 50
_N = _B * _L

_LANES = 16
_IW = 128
_CHUNK = _IW


def _make_sc_gather():
  info = plsc.get_sparse_core_info()
  nw = info.num_cores * info.num_subcores
  per_w = _N // nw
  n_irows = per_w // _IW
  n_chunks = per_w // _CHUNK

  mesh = plsc.VectorSubcoreMesh(core_axis_name="c", subcore_axis_name="s")

  @functools.partial(
      pl.kernel,
      mesh=mesh,
      out_type=jax.ShapeDtypeStruct((_N, _EMB), jnp.float32),
      scratch_types=[
          pltpu.VMEM((n_irows, _IW), jnp.int32),
          pltpu.VMEM((n_irows, _IW), jnp.int32),
          pltpu.VMEM((_CHUNK, _EMB), jnp.float32),
          pltpu.SemaphoreType.DMA,
          pltpu.SemaphoreType.DMA,
      ],
      compiler_params=pltpu.CompilerParams(
          use_tc_tiling_on_sc=False, needs_layout_passes=False),
  )
  def gather_k(src_hbm, table_hbm, out_hbm, jpos_v, idx_v, buf_v,
               sem_g, sem_i):
    wid = lax.axis_index("s") * info.num_cores + lax.axis_index("c")
    base = wid * per_w
    n_groups = _CHUNK // _LANES
    gpr = _IW // _LANES

    def pos_row(r, carry):
      def pos_group(g, carry2):
        k = (base + r * _IW + g * _LANES
             + lax.iota(jnp.int32, _LANES))
        b = lax.rem(k, _B)
        t = lax.div(k, _B)
        jpos_v[r, pl.ds(g * _LANES, _LANES)] = b * _L + t
        return carry2
      return lax.fori_loop(0, gpr, pos_group, carry)
    lax.fori_loop(0, n_irows, pos_row, 0)

    def fire_idx(r, carry):
      pltpu.async_copy(src_hbm.at[jpos_v.at[r]], idx_v.at[r], sem_i)
      return carry
    lax.fori_loop(0, n_irows, fire_idx, 0)
    def drain_idx(r, carry):
      pltpu.make_async_copy(
          src_hbm.at[jpos_v.at[r]], idx_v.at[r], sem_i).wait()
      return carry
    lax.fori_loop(0, n_irows, drain_idx, 0)

    def zero_pad_rows(c):
      def cnt_group(g, acc):
        idx16 = idx_v[c, pl.ds(g * _LANES, _LANES)]
        return acc + plsc.all_reduce_population_count(idx16 == 0)
      cnt_vec = lax.fori_loop(
          0, gpr, cnt_group, jnp.zeros((_LANES,), jnp.int32))
      cnt = jnp.sum(cnt_vec)

      @pl.when(cnt > 0)
      def _():
        def zero_group(g, carry2):
          idx16 = idx_v[c, pl.ds(g * _LANES, _LANES)]
          m = idx16 == 0
          rowids = g * _LANES + lax.iota(jnp.int32, _LANES)
          zeros16 = jnp.zeros((_LANES,), jnp.float32)
          for s in range(_EMB):
            plsc.store_scatter(
                buf_v, [rowids, jnp.full((_LANES,), s, jnp.int32)],
                zeros16, mask=m)
          return carry2
        lax.fori_loop(0, gpr, zero_group, 0)

    def chunk_body(c, carry):
      pltpu.async_copy(table_hbm.at[idx_v.at[c]], buf_v, sem_g).wait()
      zero_pad_rows(c)
      pltpu.sync_copy(buf_v, out_hbm.at[pl.ds(base + c * _CHUNK, _CHUNK)])
      return carry

    lax.fori_loop(0, n_chunks, chunk_body, 0)

  return gather_k


def _gru_body(x_ref, wxr_ref, wxz_ref, wxn_ref,
              whr_ref, whz_ref, whn_ref,
              bir_ref, biz_ref, bin_ref, bhr_ref, bhz_ref, bhn_ref,
              h_ref):
  t = pl.program_id(0)

  @pl.when(t == 0)
  def _():
    h_ref[...] = jnp.zeros_like(h_ref)

  x = x_ref[0]
  h = h_ref[...]

  def mm(a, w_ref):
    return lax.dot_general(
        a, w_ref[...], (((1,), (0,)), ((), ())),
        preferred_element_type=jnp.float32)

  gxr = mm(x, wxr_ref) + bir_ref[...]
  gxz = mm(x, wxz_ref) + biz_ref[...]
  gxn = mm(x, wxn_ref) + bin_ref[...]
  ghr = mm(h, whr_ref) + bhr_ref[...]
  ghz = mm(h, whz_ref) + bhz_ref[...]
  ghn = mm(h, whn_ref) + bhn_ref[...]

  r = jax.nn.sigmoid(gxr + ghr)
  z = jax.nn.sigmoid(gxz + ghz)
  n = jnp.tanh(gxn + r * ghn)
  h_ref[...] = (1.0 - z) * n + z * h


def _run_gru(xs, W_ih, W_hh, b_ih, b_hh):
  wih_t = W_ih.T
  whh_t = W_hh.T
  wxr, wxz, wxn = (wih_t[:, :_HID], wih_t[:, _HID:2 * _HID],
                   wih_t[:, 2 * _HID:])
  whr, whz, whn = (whh_t[:, :_HID], whh_t[:, _HID:2 * _HID],
                   whh_t[:, 2 * _HID:])
  bir, biz, bin_ = (b_ih[:_HID].reshape(1, _HID),
                    b_ih[_HID:2 * _HID].reshape(1, _HID),
                    b_ih[2 * _HID:].reshape(1, _HID))
  bhr, bhz, bhn = (b_hh[:_HID].reshape(1, _HID),
                   b_hh[_HID:2 * _HID].reshape(1, _HID),
                   b_hh[2 * _HID:].reshape(1, _HID))

  full = lambda shape: pl.BlockSpec(shape, lambda t: (0,) * len(shape))
  grid_spec = pltpu.PrefetchScalarGridSpec(
      num_scalar_prefetch=0,
      grid=(_L,),
      in_specs=[
          pl.BlockSpec((1, _B, _EMB), lambda t: (t, 0, 0)),
          full((_EMB, _HID)), full((_EMB, _HID)), full((_EMB, _HID)),
          full((_HID, _HID)), full((_HID, _HID)), full((_HID, _HID)),
          full((1, _HID)), full((1, _HID)), full((1, _HID)),
          full((1, _HID)), full((1, _HID)), full((1, _HID)),
      ],
      out_specs=pl.BlockSpec((_B, _HID), lambda t: (0, 0)),
  )
  h = pl.pallas_call(
      _gru_body,
      grid_spec=grid_spec,
      out_shape=jax.ShapeDtypeStruct((_B, _HID), jnp.float32),
  )(xs, wxr, wxz, wxn, whr, whz, whn, bir, biz, bin_, bhr, bhz, bhn)
  return h


@jax.jit
def kernel(src, emb_table, W_ih, W_hh, b_ih, b_hh):
  gather_k = _make_sc_gather()
  src_flat = src.reshape(_N)
  gathered = gather_k(src_flat, emb_table)
  xs = gathered.reshape(_L, _B, _EMB)
  h = _run_gru(xs, W_ih, W_hh, b_ih, b_hh)
  return h[None, :, :]


# speedup vs baseline: 1.1428x; 1.1053x over previous
import functools

import jax
import jax.numpy as jnp
from jax import lax
from jax.experimental import pallas as pl
from jax.experimental.pallas import tpu as pltpu
from jax.experimental.pallas import tpu_sc as plsc

_VOCAB = 1000000
_EMB = 64
_HID = 64
_B = 4096
_L = 50
_N = _B * _L

_LANES = 16
_IW = 128
_XW = 128


def _make_sc_gather():
  info = plsc.get_sparse_core_info()
  nw = info.num_cores * info.num_subcores

  mesh = plsc.VectorSubcoreMesh(core_axis_name="c", subcore_axis_name="s")

  @functools.partial(
      pl.kernel,
      mesh=mesh,
      out_type=jax.ShapeDtypeStruct((_L, _B, _XW), jnp.float32),
      scratch_types=[
          pltpu.VMEM((_IW, _L), jnp.int32),
          pltpu.VMEM((_L, _IW), jnp.int32),
          pltpu.VMEM((_IW, _EMB), jnp.float32),
          pltpu.SemaphoreType.DMA,
      ],
      compiler_params=pltpu.CompilerParams(
          use_tc_tiling_on_sc=False, needs_layout_passes=False),
  )
  def gather_k(src_hbm, table_hbm, out_hbm, srcbuf_v, idx_v, buf_v, sem_g):
    wid = lax.axis_index("s") * info.num_cores + lax.axis_index("c")
    b0 = wid * _IW
    gpr = _IW // _LANES

    pltpu.sync_copy(src_hbm.at[pl.ds(b0, _IW)], srcbuf_v)

    def trans_row(t, carry):
      def trans_group(g, carry2):
        rows = g * _LANES + lax.iota(jnp.int32, _LANES)
        cols = jnp.full((_LANES,), t, jnp.int32)
        idx_v[t, pl.ds(g * _LANES, _LANES)] = plsc.load_gather(
            srcbuf_v, [rows, cols])
        return carry2
      return lax.fori_loop(0, gpr, trans_group, carry)
    lax.fori_loop(0, _L, trans_row, 0)

    def zero_pad_rows(t):
      def cnt_group(g, acc):
        idx16 = idx_v[t, pl.ds(g * _LANES, _LANES)]
        return acc + plsc.all_reduce_population_count(idx16 == 0)
      cnt_vec = lax.fori_loop(
          0, gpr, cnt_group, jnp.zeros((_LANES,), jnp.int32))
      cnt = jnp.sum(cnt_vec)

      @pl.when(cnt > 0)
      def _():
        def zero_group(g, carry2):
          idx16 = idx_v[t, pl.ds(g * _LANES, _LANES)]
          m = idx16 == 0
          rowids = g * _LANES + lax.iota(jnp.int32, _LANES)
          zeros16 = jnp.zeros((_LANES,), jnp.float32)
          for s in range(_EMB):
            plsc.store_scatter(
                buf_v, [rowids, jnp.full((_LANES,), s, jnp.int32)],
                zeros16, mask=m)
          return carry2
        lax.fori_loop(0, gpr, zero_group, 0)

    def chunk_body(t, carry):
      pltpu.async_copy(table_hbm.at[idx_v.at[t]], buf_v, sem_g).wait()
      zero_pad_rows(t)
      pltpu.sync_copy(
          buf_v, out_hbm.at[t, pl.ds(b0, _IW), pl.ds(0, _EMB)])
      return carry

    lax.fori_loop(0, _L, chunk_body, 0)

  return gather_k


def _gru_body(x_ref, wxr_ref, wxz_ref, wxn_ref,
              whr_ref, whz_ref, whn_ref,
              bir_ref, biz_ref, bin_ref, bhr_ref, bhz_ref, bhn_ref,
              h_ref):
  t = pl.program_id(0)

  @pl.when(t == 0)
  def _():
    h_ref[...] = jnp.zeros_like(h_ref)

  xp = x_ref[0]
  lane = lax.broadcasted_iota(jnp.int32, (_B, _XW), 1)
  x = jnp.where(lane < _EMB, xp, 0.0)
  h = h_ref[...]

  def mm(a, w_ref):
    return lax.dot_general(
        a, w_ref[...], (((1,), (0,)), ((), ())),
        preferred_element_type=jnp.float32)

  gxr = mm(x, wxr_ref) + bir_ref[...]
  gxz = mm(x, wxz_ref) + biz_ref[...]
  gxn = mm(x, wxn_ref) + bin_ref[...]
  ghr = mm(h, whr_ref) + bhr_ref[...]
  ghz = mm(h, whz_ref) + bhz_ref[...]
  ghn = mm(h, whn_ref) + bhn_ref[...]

  r = jax.nn.sigmoid(gxr + ghr)
  z = jax.nn.sigmoid(gxz + ghz)
  n = jnp.tanh(gxn + r * ghn)
  h_ref[...] = (1.0 - z) * n + z * h


def _run_gru(xs, W_ih, W_hh, b_ih, b_hh):
  wih_t = W_ih.T
  wih_p = jnp.zeros((_XW, 3 * _HID), jnp.float32).at[:_EMB].set(wih_t)
  whh_t = W_hh.T
  wxr, wxz, wxn = (wih_p[:, :_HID], wih_p[:, _HID:2 * _HID],
                   wih_p[:, 2 * _HID:])
  whr, whz, whn = (whh_t[:, :_HID], whh_t[:, _HID:2 * _HID],
                   whh_t[:, 2 * _HID:])
  bir, biz, bin_ = (b_ih[:_HID].reshape(1, _HID),
                    b_ih[_HID:2 * _HID].reshape(1, _HID),
                    b_ih[2 * _HID:].reshape(1, _HID))
  bhr, bhz, bhn = (b_hh[:_HID].reshape(1, _HID),
                   b_hh[_HID:2 * _HID].reshape(1, _HID),
                   b_hh[2 * _HID:].reshape(1, _HID))

  full = lambda shape: pl.BlockSpec(shape, lambda t: (0,) * len(shape))
  grid_spec = pltpu.PrefetchScalarGridSpec(
      num_scalar_prefetch=0,
      grid=(_L,),
      in_specs=[
          pl.BlockSpec((1, _B, _XW), lambda t: (t, 0, 0)),
          full((_XW, _HID)), full((_XW, _HID)), full((_XW, _HID)),
          full((_HID, _HID)), full((_HID, _HID)), full((_HID, _HID)),
          full((1, _HID)), full((1, _HID)), full((1, _HID)),
          full((1, _HID)), full((1, _HID)), full((1, _HID)),
      ],
      out_specs=pl.BlockSpec((_B, _HID), lambda t: (0, 0)),
  )
  h = pl.pallas_call(
      _gru_body,
      grid_spec=grid_spec,
      out_shape=jax.ShapeDtypeStruct((_B, _HID), jnp.float32),
  )(xs, wxr, wxz, wxn, whr, whz, whn, bir, biz, bin_, bhr, bhz, bhn)
  return h


@jax.jit
def kernel(src, emb_table, W_ih, W_hh, b_ih, b_hh):
  gather_k = _make_sc_gather()
  xs = gather_k(src, emb_table)
  h = _run_gru(xs, W_ih, W_hh, b_ih, b_hh)
  return h[None, :, :]
